# baseline (speedup 1.0000x reference)
import jax
import jax.numpy as jnp
from jax.experimental import pallas as pl
import jax.experimental.pallas.tpu as pltpu

E = 64
TOP_K = 2
D_MODEL = 128
BATCH = 64
SEQ_LEN = 4096

CHUNK_B = 2
N_CH = BATCH // CHUNK_B
RING = 8


def _router_kernel(x_hbm, c_ref, scores_out_ref, idx_out_ref, acc_ref, *rest):
    bufs = rest[:RING]
    sems = rest[RING:]

    def copy(k):
        return pltpu.make_async_copy(
            x_hbm.at[pl.ds(k * CHUNK_B, CHUNK_B)], bufs[k % RING], sems[k % RING]
        )

    for k in range(RING):
        copy(k).start()
    for k in range(N_CH):
        copy(k).wait()
        acc_ref[pl.ds(k * CHUNK_B, CHUNK_B), :] = jnp.sum(bufs[k % RING][...], axis=1)
        if k + RING < N_CH:
            copy(k + RING).start()

    q = acc_ref[...] * (1.0 / SEQ_LEN)
    qn = jnp.sqrt(jnp.sum(q * q, axis=1, keepdims=True))
    q = q / jnp.maximum(qn, 1e-12)

    c = c_ref[...]
    cn = jnp.sqrt(jnp.sum(c * c, axis=1, keepdims=True))
    c = c / jnp.maximum(cn, 1e-12)

    scores = jax.lax.dot_general(
        q, c, (((1,), (1,)), ((), ())), preferred_element_type=jnp.float32
    )

    iota = jax.lax.broadcasted_iota(jnp.int32, (BATCH, E), 1)
    m1 = jnp.max(scores, axis=1, keepdims=True)
    i1 = jnp.min(
        jnp.where(scores == m1, iota, jnp.int32(2**30)), axis=1, keepdims=True
    )
    masked = jnp.where(iota == i1, -jnp.inf, scores)
    m2 = jnp.max(masked, axis=1, keepdims=True)
    i2 = jnp.min(
        jnp.where(masked == m2, iota, jnp.int32(2**30)), axis=1, keepdims=True
    )

    scores_out_ref[:, 0:1] = m1
    scores_out_ref[:, 1:2] = m2
    idx_out_ref[:, 0:1] = i1
    idx_out_ref[:, 1:2] = i2


@jax.jit
def kernel(x, centroids):
    top_scores, top_idx = pl.pallas_call(
        _router_kernel,
        in_specs=[
            pl.BlockSpec(memory_space=pl.ANY),
            pl.BlockSpec(memory_space=pltpu.MemorySpace.VMEM),
        ],
        out_specs=[
            pl.BlockSpec(memory_space=pltpu.MemorySpace.VMEM),
            pl.BlockSpec(memory_space=pltpu.MemorySpace.VMEM),
        ],
        out_shape=[
            jax.ShapeDtypeStruct((BATCH, TOP_K), jnp.float32),
            jax.ShapeDtypeStruct((BATCH, TOP_K), jnp.int32),
        ],
        scratch_shapes=(
            [pltpu.VMEM((BATCH, D_MODEL), jnp.float32)]
            + [pltpu.VMEM((CHUNK_B, SEQ_LEN, D_MODEL), jnp.float32) for _ in range(RING)]
            + [pltpu.SemaphoreType.DMA for _ in range(RING)]
        ),
    )(x, centroids)
    return top_scores, top_idx


# speedup vs baseline: 1.0186x; 1.0186x over previous
import jax
import jax.numpy as jnp
from jax.experimental import pallas as pl
import jax.experimental.pallas.tpu as pltpu

E = 64
TOP_K = 2
D_MODEL = 128
BATCH = 64
SEQ_LEN = 4096

CHUNK_B = 4
N_CH = BATCH // CHUNK_B
RING = 4


def _router_kernel(x_hbm, c_ref, scores_out_ref, idx_out_ref, acc_ref, *rest):
    bufs = rest[:RING]
    sems = rest[RING:]

    def copy(k):
        return pltpu.make_async_copy(
            x_hbm.at[pl.ds(k * CHUNK_B, CHUNK_B)], bufs[k % RING], sems[k % RING]
        )

    for k in range(RING):
        copy(k).start()
    for k in range(N_CH):
        copy(k).wait()
        acc_ref[pl.ds(k * CHUNK_B, CHUNK_B), :] = jnp.sum(bufs[k % RING][...], axis=1)
        if k + RING < N_CH:
            copy(k + RING).start()

    q = acc_ref[...] * (1.0 / SEQ_LEN)
    qn = jnp.sqrt(jnp.sum(q * q, axis=1, keepdims=True))
    q = q / jnp.maximum(qn, 1e-12)

    c = c_ref[...]
    cn = jnp.sqrt(jnp.sum(c * c, axis=1, keepdims=True))
    c = c / jnp.maximum(cn, 1e-12)

    scores = jax.lax.dot_general(
        q, c, (((1,), (1,)), ((), ())), preferred_element_type=jnp.float32
    )

    iota = jax.lax.broadcasted_iota(jnp.int32, (BATCH, E), 1)
    m1 = jnp.max(scores, axis=1, keepdims=True)
    i1 = jnp.min(
        jnp.where(scores == m1, iota, jnp.int32(2**30)), axis=1, keepdims=True
    )
    masked = jnp.where(iota == i1, -jnp.inf, scores)
    m2 = jnp.max(masked, axis=1, keepdims=True)
    i2 = jnp.min(
        jnp.where(masked == m2, iota, jnp.int32(2**30)), axis=1, keepdims=True
    )

    scores_out_ref[:, 0:1] = m1
    scores_out_ref[:, 1:2] = m2
    idx_out_ref[:, 0:1] = i1
    idx_out_ref[:, 1:2] = i2


@jax.jit
def kernel(x, centroids):
    top_scores, top_idx = pl.pallas_call(
        _router_kernel,
        in_specs=[
            pl.BlockSpec(memory_space=pl.ANY),
            pl.BlockSpec(memory_space=pltpu.MemorySpace.VMEM),
        ],
        out_specs=[
            pl.BlockSpec(memory_space=pltpu.MemorySpace.VMEM),
            pl.BlockSpec(memory_space=pltpu.MemorySpace.VMEM),
        ],
        out_shape=[
            jax.ShapeDtypeStruct((BATCH, TOP_K), jnp.float32),
            jax.ShapeDtypeStruct((BATCH, TOP_K), jnp.int32),
        ],
        scratch_shapes=(
            [pltpu.VMEM((BATCH, D_MODEL), jnp.float32)]
            + [pltpu.VMEM((CHUNK_B, SEQ_LEN, D_MODEL), jnp.float32) for _ in range(RING)]
            + [pltpu.SemaphoreType.DMA for _ in range(RING)]
        ),
    )(x, centroids)
    return top_scores, top_idx


# speedup vs baseline: 1.0193x; 1.0007x over previous
import jax
import jax.numpy as jnp
from jax.experimental import pallas as pl
import jax.experimental.pallas.tpu as pltpu

E = 64
TOP_K = 2
D_MODEL = 128
BATCH = 64
SEQ_LEN = 4096

CHUNK_B = 4
N_CH = BATCH // CHUNK_B
RING = 4


def _router_kernel(x_hbm, c_ref, scores_out_ref, idx_out_ref, acc_ref, *rest):
    bufs = rest[:RING]
    sems = rest[RING:]

    def copy(k):
        return pltpu.make_async_copy(
            x_hbm.at[pl.ds(k * CHUNK_B, CHUNK_B)], bufs[k % RING], sems[k % RING]
        )

    for k in range(RING):
        copy(k).start()
    for k in range(N_CH):
        copy(k).wait()
        acc_ref[pl.ds(k * CHUNK_B, CHUNK_B), :] = jnp.sum(bufs[k % RING][...], axis=1)
        if k + RING < N_CH:
            copy(k + RING).start()

    c = c_ref[...]
    cn = jnp.sqrt(jnp.sum(c * c, axis=1, keepdims=True))
    c = c / jnp.maximum(cn, 1e-12)

    q = acc_ref[...] * (1.0 / SEQ_LEN)
    qn = jnp.sqrt(jnp.sum(q * q, axis=1, keepdims=True))
    q = q / jnp.maximum(qn, 1e-12)

    scores = jax.lax.dot_general(
        q, c, (((1,), (1,)), ((), ())), preferred_element_type=jnp.float32
    )

    iota = jax.lax.broadcasted_iota(jnp.int32, (BATCH, E), 1)
    m1 = jnp.max(scores, axis=1, keepdims=True)
    i1 = jnp.min(
        jnp.where(scores == m1, iota, jnp.int32(2**30)), axis=1, keepdims=True
    )
    masked = jnp.where(iota == i1, -jnp.inf, scores)
    m2 = jnp.max(masked, axis=1, keepdims=True)
    i2 = jnp.min(
        jnp.where(masked == m2, iota, jnp.int32(2**30)), axis=1, keepdims=True
    )

    scores_out_ref[:, 0:1] = m1
    scores_out_ref[:, 1:2] = m2
    scores_out_ref[:, 2:D_MODEL] = jnp.zeros((BATCH, D_MODEL - 2), jnp.float32)
    idx_out_ref[:, 0:1] = i1
    idx_out_ref[:, 1:2] = i2
    idx_out_ref[:, 2:D_MODEL] = jnp.zeros((BATCH, D_MODEL - 2), jnp.int32)


@jax.jit
def kernel(x, centroids):
    top_scores, top_idx = pl.pallas_call(
        _router_kernel,
        in_specs=[
            pl.BlockSpec(memory_space=pl.ANY),
            pl.BlockSpec(memory_space=pltpu.MemorySpace.VMEM),
        ],
        out_specs=[
            pl.BlockSpec(memory_space=pltpu.MemorySpace.VMEM),
            pl.BlockSpec(memory_space=pltpu.MemorySpace.VMEM),
        ],
        out_shape=[
            jax.ShapeDtypeStruct((BATCH, D_MODEL), jnp.float32),
            jax.ShapeDtypeStruct((BATCH, D_MODEL), jnp.int32),
        ],
        scratch_shapes=(
            [pltpu.VMEM((BATCH, D_MODEL), jnp.float32)]
            + [pltpu.VMEM((CHUNK_B, SEQ_LEN, D_MODEL), jnp.float32) for _ in range(RING)]
            + [pltpu.SemaphoreType.DMA for _ in range(RING)]
        ),
    )(x, centroids)
    return top_scores[:, :TOP_K], top_idx[:, :TOP_K]
